# baseline (speedup 1.0000x reference)
import functools

import numpy as np
import jax
from jax import lax
import jax.numpy as jnp
from jax.experimental import pallas as pl
from jax.experimental.pallas import tpu as pltpu
from jax.experimental.pallas import tpu_sc as plsc

_MAX_WAVELENGTH = 10000.0


def _sine_pe_np(seq_len: int, dim: int) -> np.ndarray:
    position = np.arange(seq_len, dtype=np.float64)
    min_freq = 1.0 / _MAX_WAVELENGTH
    timescales = np.power(
        min_freq,
        (2 * (np.arange(dim) // 2)).astype(np.float64) / float(dim),
    )
    angles = position[:, None] * timescales[None, :]
    cos_mask = (np.arange(dim) % 2).astype(np.float64)
    pe = np.sin(angles) * (1.0 - cos_mask) + np.cos(angles) * cos_mask
    return pe.astype(np.float32)


def _make_sc_kernel(batch, seq_len, dim, nb):
    info = plsc.get_sparse_core_info()
    nc, ns, lanes = info.num_cores, info.num_subcores, info.num_lanes
    nw = nc * ns
    per_w = batch // nw
    n_chunks = per_w // nb
    mesh = plsc.VectorSubcoreMesh(core_axis_name="c", subcore_axis_name="s")

    @functools.partial(
        pl.kernel,
        mesh=mesh,
        out_type=jax.ShapeDtypeStruct((batch, seq_len, dim), jnp.float32),
        scratch_types=[
            pltpu.VMEM((nb, seq_len, dim), jnp.float32),
            pltpu.VMEM((seq_len, dim), jnp.float32),
        ],
    )
    def sc_add(x_hbm, pe_hbm, out_hbm, buf, pe_v):
        wid = lax.axis_index("s") * nc + lax.axis_index("c")
        base = wid * per_w
        pltpu.sync_copy(pe_hbm, pe_v)

        def chunk(k, carry):
            b0 = base + k * nb
            pltpu.sync_copy(x_hbm.at[pl.ds(b0, nb)], buf)
            for si in range(seq_len):
                for j in range(dim // lanes):
                    pe_vec = pe_v[si, pl.ds(j * lanes, lanes)]

                    def badd(b, pv):
                        buf[b, si, pl.ds(j * lanes, lanes)] = (
                            buf[b, si, pl.ds(j * lanes, lanes)] + pv)
                        return pv

                    lax.fori_loop(0, nb, badd, pe_vec)
            pltpu.sync_copy(buf, out_hbm.at[pl.ds(b0, nb)])
            return carry

        lax.fori_loop(0, n_chunks, chunk, 0)

    return sc_add


def kernel(inputs, table):
    batch, seq_len, dim = inputs.shape
    pe = jnp.asarray(_sine_pe_np(seq_len, dim))
    return _make_sc_kernel(batch, seq_len, dim, nb=8)(inputs, pe)
